# baseline (speedup 1.0000x reference)
import functools

import jax
import jax.numpy as jnp
from jax import lax
from jax.experimental import pallas as pl
from jax.experimental.pallas import tpu as pltpu
from jax.experimental.pallas import tpu_sc as plsc

_HIDDEN = 64
_RPC = 4
_LANES = 16
_SUB = 128


def _xpose_block(in_hbm, out_ref, vb0, vb1, sem0, sem1):
    lo, ho, _, hi, bb = out_ref.shape
    lh = lo * ho * hi
    i = pl.program_id(0)
    nblk = pl.num_programs(0)

    def start(blk, vb, sem):
        base = blk * (bb * lh)
        for r in range(bb):
            pltpu.make_async_copy(
                in_hbm.at[pl.ds(base + r * lh, lh)], vb.at[r], sem
            ).start()

    def compute(vb, sem):
        for r in range(bb):
            pltpu.make_async_copy(
                in_hbm.at[pl.ds(r * lh, lh)], vb.at[r], sem
            ).wait()
        t = jnp.transpose(vb[...].reshape(bb, lh // bb, bb), (1, 2, 0))
        out_ref[...] = t.reshape(lo, ho, hi, bb)[:, :, None, :, :]

    even = (i % 2) == 0

    @pl.when(i == 0)
    def _():
        start(0, vb0, sem0)

    @pl.when((i + 1 < nblk) & even)
    def _():
        start(i + 1, vb1, sem1)

    @pl.when((i + 1 < nblk) & jnp.logical_not(even))
    def _():
        start(i + 1, vb0, sem0)

    @pl.when(even)
    def _():
        compute(vb0, sem0)

    @pl.when(jnp.logical_not(even))
    def _():
        compute(vb1, sem1)


@functools.lru_cache(maxsize=None)
def _make_xpose(b, l, h):
    bb = 128
    nblk = b // bb
    return pl.pallas_call(
        _xpose_block,
        grid=(nblk,),
        in_specs=[pl.BlockSpec(memory_space=pl.ANY)],
        out_specs=pl.BlockSpec(
            (l, h // 8, 1, 8, bb), lambda i: (0, 0, i, 0, 0)),
        out_shape=jax.ShapeDtypeStruct((l, h // 8, nblk, 8, bb), jnp.float32),
        scratch_shapes=[
            pltpu.VMEM((bb, l * h), jnp.float32),
            pltpu.VMEM((bb, l * h), jnp.float32),
            pltpu.SemaphoreType.DMA,
            pltpu.SemaphoreType.DMA,
        ],
    )


@functools.lru_cache(maxsize=None)
def _make_kernel(b, l):
    info = plsc.get_sparse_core_info()
    nc, ns = info.num_cores, info.num_subcores
    nw = nc * ns
    bb = b // nw
    assert bb == 128 and l % 2 == 0

    mesh = plsc.VectorSubcoreMesh(core_axis_name="c", subcore_axis_name="s")

    @functools.partial(
        pl.kernel,
        mesh=mesh,
        compiler_params=pltpu.CompilerParams(
            needs_layout_passes=False, use_tc_tiling_on_sc=False
        ),
        out_type=jax.ShapeDtypeStruct((b, l, _HIDDEN), jnp.float32),
        scratch_types=[
            pltpu.VMEM((2, 2, 128), jnp.float32),
            pltpu.VMEM((2 * 128,), jnp.int32),
            pltpu.VMEM((2 * 128, 1, _HIDDEN), jnp.float32),
            pltpu.SemaphoreType.DMA,
            pltpu.SemaphoreType.DMA,
            pltpu.SemaphoreType.DMA,
            pltpu.SemaphoreType.DMA,
            pltpu.SemaphoreType.DMA,
            pltpu.SemaphoreType.DMA,
        ],
    )
    def k(x5_hbm, table_hbm, out_hbm, xbuf, idxbuf, rowsbuf,
          sin0, sin1, sg0, sg1, sout0, sout1):
        sin = (sin0, sin1)
        sg = (sg0, sg1)
        sout = (sout0, sout1)
        wid = lax.axis_index("s") * nc + lax.axis_index("c")
        b0 = wid * 128
        iota = lax.iota(jnp.int32, _LANES)
        zer = jnp.zeros((_LANES,), jnp.int32)
        c63 = jnp.full((_LANES,), _HIDDEN - 1, jnp.int32)

        def in_x(g, p):
            return pltpu.make_async_copy(
                x5_hbm.at[g, wid], xbuf.at[p], sin[p])

        def convert(p):
            for j in range(128 // _LANES):
                v = xbuf[p, 0, pl.ds(j * _LANES, _LANES)]
                idxbuf[pl.ds(p * 128 + j * _LANES, _LANES)] = (
                    v.astype(jnp.int32))

        def gather(p):
            return pltpu.make_async_copy(
                table_hbm.at[idxbuf.at[pl.ds(p * 128, 128)]],
                rowsbuf.at[pl.ds(p * 128, 128), :, :],
                sg[p],
            )

        def durscatter(p):
            for j in range(128 // _LANES):
                r = p * 128 + j * _LANES + iota
                durv = xbuf[p, 1, pl.ds(j * _LANES, _LANES)]
                plsc.store_scatter(rowsbuf, [r, zer, c63], durv)

        def out_copy(g, p):
            return pltpu.make_async_copy(
                rowsbuf.at[pl.ds(p * 128, 128), :, :],
                out_hbm.at[pl.ds(b0, 128), pl.ds(g, 1), :],
                sout[p],
            )

        in_x(0, 0).start()
        in_x(0, 0).wait()
        convert(0)
        gather(0).start()
        in_x(1, 1).start()

        def sub_body(g, p):
            pn = 1 - p

            @pl.when(g + 1 < l)
            def _():
                in_x(g + 1, pn).wait()
                convert(pn)

            gather(p).wait()
            durscatter(p)

            @pl.when(g + 2 < l)
            def _():
                in_x(g + 2, p).start()

            @pl.when(g >= 1)
            def _():
                out_copy(g - 1, pn).wait()

            @pl.when(g + 1 < l)
            def _():
                gather(pn).start()

            out_copy(g, p).start()

        def macro(t, carry):
            sub_body(2 * t, 0)
            sub_body(2 * t + 1, 1)
            return carry

        lax.fori_loop(0, l // 2, macro, 0)
        out_copy(l - 1, 1).wait()

    return k


def kernel(x, table):
    b, l, _ = x.shape
    n = b * l
    x5 = (x.transpose(1, 2, 0)
           .reshape(l, 2, b // 128, 128)
           .transpose(0, 2, 1, 3))
    table_pad = jnp.pad(table, ((0, 0), (0, _HIDDEN - table.shape[1])))
    table_pad = table_pad.reshape(table.shape[0], 1, _HIDDEN)
    sc_out = _make_kernel(b, l)(x5, table_pad)
    flat = sc_out.reshape(n * _HIDDEN)
    out5 = _make_xpose(b, l, _HIDDEN)(flat)
    return jnp.transpose(out5, (2, 4, 0, 1, 3)).reshape(b, l, _HIDDEN)


# speedup vs baseline: 2.0264x; 2.0264x over previous
import functools

import jax
import jax.numpy as jnp
from jax import lax
from jax.experimental import pallas as pl
from jax.experimental.pallas import tpu as pltpu
from jax.experimental.pallas import tpu_sc as plsc

_HIDDEN = 64
_RPC = 4
_LANES = 16
_SUB = 128


def _xpose_block(in_hbm, out_ref, vb0, vb1, sem0, sem1):
    lo, ho, _, hi, bb = out_ref.shape
    lh = lo * ho * hi
    i = pl.program_id(0)
    nblk = pl.num_programs(0)

    def start(blk, vb, sem):
        base = blk * (bb * lh)
        for r in range(bb):
            pltpu.make_async_copy(
                in_hbm.at[pl.ds(base + r * lh, lh)], vb.at[r], sem
            ).start()

    def compute(vb, sem):
        for r in range(bb):
            pltpu.make_async_copy(
                in_hbm.at[pl.ds(r * lh, lh)], vb.at[r], sem
            ).wait()
        t = jnp.transpose(vb[...].reshape(bb, lh // bb, bb), (1, 2, 0))
        out_ref[...] = t.reshape(lo, ho, hi, bb)[:, :, None, :, :]

    even = (i % 2) == 0

    @pl.when(i == 0)
    def _():
        start(0, vb0, sem0)

    @pl.when((i + 1 < nblk) & even)
    def _():
        start(i + 1, vb1, sem1)

    @pl.when((i + 1 < nblk) & jnp.logical_not(even))
    def _():
        start(i + 1, vb0, sem0)

    @pl.when(even)
    def _():
        compute(vb0, sem0)

    @pl.when(jnp.logical_not(even))
    def _():
        compute(vb1, sem1)


@functools.lru_cache(maxsize=None)
def _make_xpose(b, l, h):
    bb = 128
    nblk = b // bb
    return pl.pallas_call(
        _xpose_block,
        grid=(nblk,),
        in_specs=[pl.BlockSpec(memory_space=pl.ANY)],
        out_specs=pl.BlockSpec(
            (l, h // 8, 1, 8, bb), lambda i: (0, 0, i, 0, 0)),
        out_shape=jax.ShapeDtypeStruct((l, h // 8, nblk, 8, bb), jnp.float32),
        scratch_shapes=[
            pltpu.VMEM((bb, l * h), jnp.float32),
            pltpu.VMEM((bb, l * h), jnp.float32),
            pltpu.SemaphoreType.DMA,
            pltpu.SemaphoreType.DMA,
        ],
    )


@functools.lru_cache(maxsize=None)
def _make_kernel(b, l):
    n_rows = b * l
    chunk = _RPC * l
    info = plsc.get_sparse_core_info()
    nc, ns = info.num_cores, info.num_subcores
    nw = nc * ns
    per_w = n_rows // nw
    n_chunks = per_w // chunk
    assert per_w * nw == n_rows and n_chunks * chunk == per_w
    assert n_chunks % 2 == 0 and n_chunks >= 4
    subs = []
    off = 0
    while off < chunk:
        sz = min(_SUB, chunk - off)
        subs.append((off, sz))
        off += sz

    mesh = plsc.VectorSubcoreMesh(core_axis_name="c", subcore_axis_name="s")

    @functools.partial(
        pl.kernel,
        mesh=mesh,
        compiler_params=pltpu.CompilerParams(
            needs_layout_passes=False, use_tc_tiling_on_sc=False
        ),
        out_type=jax.ShapeDtypeStruct((b, l, _HIDDEN), jnp.float32),
        scratch_types=[
            pltpu.VMEM((2 * chunk,), jnp.int32),
            pltpu.VMEM((2 * chunk,), jnp.float32),
            pltpu.VMEM((2 * chunk, _HIDDEN), jnp.float32),
            pltpu.SemaphoreType.DMA,
            pltpu.SemaphoreType.DMA,
            pltpu.SemaphoreType.DMA,
            pltpu.SemaphoreType.DMA,
            pltpu.SemaphoreType.DMA,
            pltpu.SemaphoreType.DMA,
        ],
    )
    def k(idx_hbm, dur_hbm, table_hbm, out_hbm, idxbuf, durbuf, rowsbuf,
          sin0, sin1, sg0, sg1, sout0, sout1):
        sin = (sin0, sin1)
        sg = (sg0, sg1)
        sout = (sout0, sout1)
        wid = lax.axis_index("s") * nc + lax.axis_index("c")
        base = wid * per_w
        rbase = wid * (b // nw)
        iota = lax.iota(jnp.int32, _LANES)
        c63 = jnp.full((_LANES,), _HIDDEN - 1, jnp.int32)

        def in_idx(g, p):
            return pltpu.make_async_copy(
                idx_hbm.at[pl.ds(base + g * chunk, chunk)],
                idxbuf.at[pl.ds(p * chunk, chunk)],
                sin[p],
            )

        def in_dur(g, p):
            return pltpu.make_async_copy(
                dur_hbm.at[pl.ds(base + g * chunk, chunk)],
                durbuf.at[pl.ds(p * chunk, chunk)],
                sin[p],
            )

        def fire_gathers(p):
            for s_off, s_sz in subs:
                pltpu.async_copy(
                    table_hbm.at[idxbuf.at[pl.ds(p * chunk + s_off, s_sz)]],
                    rowsbuf.at[pl.ds(p * chunk + s_off, s_sz), :],
                    sg[p],
                )

        def wait_gathers(p):
            pltpu.make_async_copy(
                table_hbm.at[idxbuf.at[pl.ds(p * chunk, chunk)]],
                rowsbuf.at[pl.ds(p * chunk, chunk), :],
                sg[p],
            ).wait()

        def durscatter(p):
            for j in range(chunk // _LANES):
                r = p * chunk + j * _LANES + iota
                durv = durbuf[pl.ds(p * chunk + j * _LANES, _LANES)]
                plsc.store_scatter(rowsbuf, [r, c63], durv)

        def out_copies(g, p):
            return [
                pltpu.make_async_copy(
                    rowsbuf.at[pl.ds(p * chunk + r * l, l), :],
                    out_hbm.at[rbase + g * _RPC + r],
                    sout[p],
                )
                for r in range(_RPC)
            ]

        in_idx(0, 0).start()
        in_dur(0, 0).start()
        in_idx(0, 0).wait()
        in_dur(0, 0).wait()
        fire_gathers(0)
        in_idx(1, 1).start()
        in_dur(1, 1).start()

        def sub_body(g, p):
            pn = 1 - p

            @pl.when(g + 1 < n_chunks)
            def _():
                in_idx(g + 1, pn).wait()
                in_dur(g + 1, pn).wait()

            wait_gathers(p)
            durscatter(p)

            @pl.when(g + 2 < n_chunks)
            def _():
                in_idx(g + 2, p).start()
                in_dur(g + 2, p).start()

            @pl.when(g >= 1)
            def _():
                for c in out_copies(g - 1, pn):
                    c.wait()

            @pl.when(g + 1 < n_chunks)
            def _():
                fire_gathers(pn)

            for c in out_copies(g, p):
                c.start()

        def macro(t, carry):
            sub_body(2 * t, 0)
            sub_body(2 * t + 1, 1)
            return carry

        lax.fori_loop(0, n_chunks // 2, macro, 0)
        for c in out_copies(n_chunks - 1, 1):
            c.wait()

    return k


def kernel(x, table):
    b, l, _ = x.shape
    n = b * l
    idx = x[..., 0].astype(jnp.int32).reshape(n)
    dur = x[..., 1].reshape(n)
    table_pad = jnp.pad(table, ((0, 0), (0, _HIDDEN - table.shape[1])))
    sc_out = _make_kernel(b, l)(idx, dur, table_pad)
    flat = sc_out.reshape(n * _HIDDEN)
    out5 = _make_xpose(b, l, _HIDDEN)(flat)
    return jnp.transpose(out5, (2, 4, 0, 1, 3)).reshape(b, l, _HIDDEN)
